# Initial kernel scaffold; baseline (speedup 1.0000x reference)
import jax
import jax.numpy as jnp
from jax.experimental import pallas as pl


def kernel(words, emb, wih_t, whh_t, bg, wout_t, bout, h0, c0):
    raise NotImplementedError("write your pallas kernel here")


# speedup vs baseline: 1.0977x; 1.0977x over previous
import jax
import jax.numpy as jnp
from jax.experimental import pallas as pl
from jax.experimental.pallas import tpu as pltpu


def _lstm_kernel(words_ref,
                 emb_hbm,
                 wih_hbm,
                 whh_hbm,
                 wout_hbm,
                 bg_ref,
                 bout_ref,
                 state_ref,
                 scores_ref,
                 state_out_ref,
                 xbuf,
                 wih_v,
                 whh_v,
                 wout_v,
                 xproj,
                 hbuf,
                 sem_rows, sem_wih, sem_whh, sem_wout):
    T = scores_ref.shape[0]
    H = state_ref.shape[-1]

    row_cps = [
        pltpu.make_async_copy(emb_hbm.at[pl.ds(words_ref[t], 1), :],
                              xbuf.at[pl.ds(t, 1), :], sem_rows)
        for t in range(T)
    ]
    for cp in row_cps:
        cp.start()
    wih_cp = pltpu.make_async_copy(wih_hbm, wih_v, sem_wih)
    wih_cp.start()
    whh_cp = pltpu.make_async_copy(whh_hbm, whh_v, sem_whh)
    whh_cp.start()
    wout_cp = pltpu.make_async_copy(wout_hbm, wout_v, sem_wout)
    wout_cp.start()

    for cp in row_cps:
        cp.wait()
    wih_cp.wait()

    xproj[...] = (jnp.dot(xbuf[...], wih_v[...],
                          preferred_element_type=jnp.float32) + bg_ref[...])

    whh_cp.wait()

    h = state_ref[0:1, :]
    c = state_ref[1:2, :]
    for t in range(T):
        gates = xproj[t:t + 1, :] + jnp.dot(h, whh_v[...],
                                            preferred_element_type=jnp.float32)
        sig_if = jax.nn.sigmoid(gates[:, 0 * H:2 * H])
        g_g = jnp.tanh(gates[:, 2 * H:3 * H])
        o_g = jax.nn.sigmoid(gates[:, 3 * H:4 * H])
        c = sig_if[:, H:2 * H] * c + sig_if[:, 0:H] * g_g
        h = o_g * jnp.tanh(c)
        hbuf[t:t + 1, :] = h

    wout_cp.wait()
    logits = (jnp.dot(hbuf[...], wout_v[...],
                      preferred_element_type=jnp.float32) + bout_ref[...])
    scores_ref[...] = jax.nn.sigmoid(logits)

    state_out_ref[0:1, :] = h
    state_out_ref[1:2, :] = c


@jax.jit
def kernel(words, emb, wih_t, whh_t, bg, wout_t, bout, h0, c0):
    V, E = emb.shape
    H = h0.shape[-1]
    T = words.shape[0]

    state2 = jnp.concatenate(
        [h0.reshape(1, H), c0.reshape(1, H)], axis=0).astype(jnp.float32)
    words_i32 = jnp.clip(words.astype(jnp.int32), 0, V - 1)

    vmem = lambda i, w: (0, 0)
    any_spec = pl.BlockSpec(memory_space=pl.ANY)

    scores, state_out = pl.pallas_call(
        _lstm_kernel,
        out_shape=(
            jax.ShapeDtypeStruct((T, V), jnp.float32),
            jax.ShapeDtypeStruct((2, H), jnp.float32),
        ),
        grid_spec=pltpu.PrefetchScalarGridSpec(
            num_scalar_prefetch=1,
            grid=(1,),
            in_specs=[
                any_spec,
                any_spec,
                any_spec,
                any_spec,
                pl.BlockSpec((1, 4 * H), vmem),
                pl.BlockSpec((1, V), vmem),
                pl.BlockSpec((2, H), vmem),
            ],
            out_specs=[
                pl.BlockSpec((T, V), vmem),
                pl.BlockSpec((2, H), vmem),
            ],
            scratch_shapes=[
                pltpu.VMEM((T, E), jnp.float32),
                pltpu.VMEM((E, 4 * H), jnp.float32),
                pltpu.VMEM((H, 4 * H), jnp.float32),
                pltpu.VMEM((H, V), jnp.float32),
                pltpu.VMEM((T, 4 * H), jnp.float32),
                pltpu.VMEM((T, H), jnp.float32),
                pltpu.SemaphoreType.DMA,
                pltpu.SemaphoreType.DMA,
                pltpu.SemaphoreType.DMA,
                pltpu.SemaphoreType.DMA,
            ],
        ),
        compiler_params=pltpu.CompilerParams(
            dimension_semantics=("arbitrary",)),
    )(words_i32, emb, wih_t, whh_t, wout_t, bg, bout, state2)

    h_new = state_out[0:1, :].reshape(1, 1, H)
    c_new = state_out[1:2, :].reshape(1, 1, H)
    return scores, (h_new, c_new)


# speedup vs baseline: 1.1028x; 1.0047x over previous
import jax
import jax.numpy as jnp
from jax.experimental import pallas as pl
from jax.experimental.pallas import tpu as pltpu


def _lstm_kernel(words_ref,
                 emb_hbm,
                 wih_hbm,
                 whh_hbm,
                 wout_hbm,
                 bg_ref,
                 bout_ref,
                 state_ref,
                 scores_ref,
                 state_out_ref,
                 xbuf,
                 wih_v,
                 whh_v,
                 wout_v,
                 whh16_s,
                 xproj,
                 hbuf,
                 sem_rows, sem_wih, sem_whh, sem_wout):
    T = scores_ref.shape[0]
    H = state_ref.shape[-1]

    row_cps = [
        pltpu.make_async_copy(emb_hbm.at[pl.ds(words_ref[t], 1), :],
                              xbuf.at[pl.ds(t, 1), :], sem_rows)
        for t in range(T)
    ]
    for cp in row_cps:
        cp.start()
    wih_cp = pltpu.make_async_copy(wih_hbm, wih_v, sem_wih)
    wih_cp.start()
    whh_cp = pltpu.make_async_copy(whh_hbm, whh_v, sem_whh)
    whh_cp.start()
    wout_cp = pltpu.make_async_copy(wout_hbm, wout_v, sem_wout)
    wout_cp.start()

    for cp in row_cps:
        cp.wait()
    wih_cp.wait()

    xproj[...] = (jnp.dot(xbuf[...], wih_v[...],
                          preferred_element_type=jnp.float32) + bg_ref[...])

    whh_cp.wait()
    whh16_s[...] = whh_v[...].astype(jnp.bfloat16)

    h = state_ref[0:1, :]
    c = state_ref[1:2, :]
    for t in range(T):
        h16 = h.astype(jnp.bfloat16)
        gates = xproj[t:t + 1, :] + jnp.dot(h16, whh16_s[...],
                                            preferred_element_type=jnp.float32)
        sig_if = jax.nn.sigmoid(gates[:, 0 * H:2 * H])
        g_g = jnp.tanh(gates[:, 2 * H:3 * H])
        o_g = jax.nn.sigmoid(gates[:, 3 * H:4 * H])
        c = sig_if[:, H:2 * H] * c + sig_if[:, 0:H] * g_g
        h = o_g * jnp.tanh(c)
        hbuf[t:t + 1, :] = h

    wout_cp.wait()
    logits = (jnp.dot(hbuf[...], wout_v[...],
                      preferred_element_type=jnp.float32) + bout_ref[...])
    scores_ref[...] = jax.nn.sigmoid(logits)

    state_out_ref[0:1, :] = h
    state_out_ref[1:2, :] = c


@jax.jit
def kernel(words, emb, wih_t, whh_t, bg, wout_t, bout, h0, c0):
    V, E = emb.shape
    H = h0.shape[-1]
    T = words.shape[0]

    state2 = jnp.concatenate(
        [h0.reshape(1, H), c0.reshape(1, H)], axis=0).astype(jnp.float32)
    words_i32 = jnp.clip(words.astype(jnp.int32), 0, V - 1)

    vmem = lambda i, w: (0, 0)
    any_spec = pl.BlockSpec(memory_space=pl.ANY)

    scores, state_out = pl.pallas_call(
        _lstm_kernel,
        out_shape=(
            jax.ShapeDtypeStruct((T, V), jnp.float32),
            jax.ShapeDtypeStruct((2, H), jnp.float32),
        ),
        grid_spec=pltpu.PrefetchScalarGridSpec(
            num_scalar_prefetch=1,
            grid=(1,),
            in_specs=[
                any_spec,
                any_spec,
                any_spec,
                any_spec,
                pl.BlockSpec((1, 4 * H), vmem),
                pl.BlockSpec((1, V), vmem),
                pl.BlockSpec((2, H), vmem),
            ],
            out_specs=[
                pl.BlockSpec((T, V), vmem),
                pl.BlockSpec((2, H), vmem),
            ],
            scratch_shapes=[
                pltpu.VMEM((T, E), jnp.float32),
                pltpu.VMEM((E, 4 * H), jnp.float32),
                pltpu.VMEM((H, 4 * H), jnp.float32),
                pltpu.VMEM((H, V), jnp.float32),
                pltpu.VMEM((H, 4 * H), jnp.bfloat16),
                pltpu.VMEM((T, 4 * H), jnp.float32),
                pltpu.VMEM((T, H), jnp.float32),
                pltpu.SemaphoreType.DMA,
                pltpu.SemaphoreType.DMA,
                pltpu.SemaphoreType.DMA,
                pltpu.SemaphoreType.DMA,
            ],
        ),
        compiler_params=pltpu.CompilerParams(
            dimension_semantics=("arbitrary",)),
    )(words_i32, emb, wih_t, whh_t, wout_t, bg, bout, state2)

    h_new = state_out[0:1, :].reshape(1, 1, H)
    c_new = state_out[1:2, :].reshape(1, 1, H)
    return scores, (h_new, c_new)
